# baseline (speedup 1.0000x reference)
import functools
import math

import jax
import jax.numpy as jnp
from jax import lax
from jax.experimental import pallas as pl
from jax.experimental.pallas import tpu as pltpu

N = 10000
D = 128
E = 320000
KS = (2000, 1000, 500)
f32 = jnp.float32
i32 = jnp.int32


def _stats_combine_body(stats_ref, x_ref, w_ref, g_ref, dinv_ref, selfw_ref):
    indeg = stats_ref[0, 0, :N] + stats_ref[1, 0, :N]
    selfcnt = stats_ref[0, 1, :N] + stats_ref[1, 1, :N]
    selfw = jnp.where(selfcnt > 0.0, 0.0, 2.0)
    deg = indeg + selfw
    dinv = jax.lax.rsqrt(jnp.maximum(deg, 1e-30))
    dinv_ref[:, 0] = dinv
    selfw_ref[:, 0] = selfw
    h = jnp.dot(x_ref[...], w_ref[...], preferred_element_type=f32)
    g_ref[...] = dinv[:, None] * h


def stats_combine_premul(statsP, x, W):
    return pl.pallas_call(
        _stats_combine_body,
        out_shape=(
            jax.ShapeDtypeStruct((N, D), f32),
            jax.ShapeDtypeStruct((N, 1), f32),
            jax.ShapeDtypeStruct((N, 1), f32),
        ),
    )(statsP, x, W)


def _premul_body(x_ref, w_ref, dinv_ref, g_ref):
    h = jnp.dot(x_ref[...], w_ref[...], preferred_element_type=f32)
    g_ref[...] = dinv_ref[:, 0][:, None] * h


def premul(x, W, dinv):
    return pl.pallas_call(
        _premul_body,
        out_shape=jax.ShapeDtypeStruct((N, D), f32),
    )(x, W, dinv)


def _gcn0_post_body(yP_ref, g_ref, dinv_ref, selfw_ref, b_ref, p_ref,
                    x1_ref, s_ref):
    y = yP_ref[0, :N, :] + yP_ref[1, :N, :]
    g = g_ref[...]
    dinv = dinv_ref[:, 0][:, None]
    out = dinv * (y + selfw_ref[:, 0][:, None] * g) + b_ref[0, :][None, :]
    x1 = jnp.maximum(out, 0.0)
    x1_ref[...] = x1
    s_ref[:, 0] = jnp.tanh(jnp.dot(x1, p_ref[:, 0], preferred_element_type=f32))


def gcn0_post(yP, g, dinv, selfw, b, pn):
    return pl.pallas_call(
        _gcn0_post_body,
        out_shape=(
            jax.ShapeDtypeStruct((N, D), f32),
            jax.ShapeDtypeStruct((N, 1), f32),
        ),
    )(yP, g, dinv, selfw, b.reshape(1, D), pn.reshape(D, 1))


def _final_body(yP_ref, g_ref, dinv_ref, selfw_ref, c_ref, out_ref):
    y = yP_ref[0, :N, :] + yP_ref[1, :N, :]
    dinv = dinv_ref[:, 0][:, None]
    out = dinv * (y + selfw_ref[:, 0][:, None] * g_ref[...]) + c_ref[0, :][None, :]
    out_ref[...] = jax.nn.sigmoid(out)


def final_post(yP, g, dinv, selfw, c):
    return pl.pallas_call(
        _final_body,
        out_shape=jax.ShapeDtypeStruct((N, D), f32),
    )(yP, g, dinv, selfw, c.reshape(1, D))


def _bigmm_body(ct_ref, b_ref, o_ref):
    k = pl.program_id(2)

    @pl.when(k == 0)
    def _():
        o_ref[...] = jnp.zeros_like(o_ref)

    o_ref[...] += lax.dot_general(
        ct_ref[...], b_ref[...], (((0,), (0,)), ((), ())),
        preferred_element_type=f32)

    @pl.when(k == pl.num_programs(2) - 1)
    def _():
        i = pl.program_id(0)
        j = pl.program_id(1)
        rows = i * o_ref.shape[0] + lax.broadcasted_iota(i32, o_ref.shape, 0)
        cols = j * o_ref.shape[1] + lax.broadcasted_iota(i32, o_ref.shape, 1)
        o_ref[...] = jnp.where(rows == cols, 0.0, o_ref[...])


def bigmm_diag0(CT, B, npad, kpad, bk=1024, bo=512):
    grid = (kpad // bo, kpad // bo, npad // bk)
    return pl.pallas_call(
        _bigmm_body,
        grid=grid,
        in_specs=[
            pl.BlockSpec((bk, bo), lambda i, j, k: (k, i)),
            pl.BlockSpec((bk, bo), lambda i, j, k: (k, j)),
        ],
        out_specs=pl.BlockSpec((bo, bo), lambda i, j, k: (i, j)),
        out_shape=jax.ShapeDtypeStruct((kpad, kpad), f32),
    )(CT, B)


def _gcnd_body(relu, score, a_ref, x_ref, w_ref, b_ref, vals_ref, p_ref,
               o_ref, s_ref):
    A = a_ref[...]
    xin = x_ref[...] * vals_ref[:, 0][:, None]
    deg = jnp.sum(A, axis=1) + 2.0
    dinv = jax.lax.rsqrt(deg)
    h = jnp.dot(xin, w_ref[...], preferred_element_type=f32)
    g = dinv[:, None] * h
    y = jnp.dot(A, g, preferred_element_type=f32)
    out = dinv[:, None] * y + 2.0 * (dinv * dinv)[:, None] * h + b_ref[0, :][None, :]
    if relu:
        out = jnp.maximum(out, 0.0)
    o_ref[...] = out
    if score:
        s_ref[:, 0] = jnp.tanh(jnp.dot(out, p_ref[:, 0], preferred_element_type=f32))


def gcn_dense(A, xin, W, b, vals=None, pn=None, relu=True):
    n = A.shape[0]
    if vals is None:
        vals = jnp.ones((n, 1), f32)
    score = pn is not None
    if pn is None:
        pn = jnp.zeros((D, 1), f32)
    out, s = pl.pallas_call(
        functools.partial(_gcnd_body, relu, score),
        out_shape=(
            jax.ShapeDtypeStruct((n, D), f32),
            jax.ShapeDtypeStruct((n, 1), f32),
        ),
    )(A, xin, W, b.reshape(1, D), vals, pn.reshape(D, 1))
    return out, s


def _aug_body(a_ref, perm_ref, x_ref, vals_ref, ap_ref, xp_ref):
    n = a_ref.shape[0]
    k = ap_ref.shape[0]
    A = a_ref[...]
    ii = lax.broadcasted_iota(i32, (n, k), 0)
    S = (ii == perm_ref[0, :][None, :]).astype(f32)
    T1 = jnp.dot(A, S, preferred_element_type=f32) + S
    T2 = jnp.dot(A, T1, preferred_element_type=f32) + T1
    Ap = lax.dot_general(S, T2, (((0,), (0,)), ((), ())),
                         preferred_element_type=f32)
    ri = lax.broadcasted_iota(i32, (k, k), 0)
    ci = lax.broadcasted_iota(i32, (k, k), 1)
    ap_ref[...] = jnp.where(ri == ci, 0.0, Ap)
    xp = lax.dot_general(S, x_ref[...], (((0,), (0,)), ((), ())),
                         preferred_element_type=f32)
    xp_ref[...] = xp * vals_ref[:, 0][:, None]


def aug_pool(A, perm, x, vals):
    n = A.shape[0]
    k = perm.shape[0]
    return pl.pallas_call(
        _aug_body,
        out_shape=(
            jax.ShapeDtypeStruct((k, k), f32),
            jax.ShapeDtypeStruct((k, D), f32),
        ),
    )(A, perm.reshape(1, k), x, vals.reshape(k, 1))


def _upgcn_body(relu, a_ref, xs_ref, u_ref, perm_ref, w_ref, b_ref, o_ref):
    n = a_ref.shape[0]
    k = u_ref.shape[0]
    ii = lax.broadcasted_iota(i32, (n, k), 0)
    S = (ii == perm_ref[0, :][None, :]).astype(f32)
    xin = xs_ref[...] + jnp.dot(S, u_ref[...], preferred_element_type=f32)
    A = a_ref[...]
    deg = jnp.sum(A, axis=1) + 2.0
    dinv = jax.lax.rsqrt(deg)
    h = jnp.dot(xin, w_ref[...], preferred_element_type=f32)
    g = dinv[:, None] * h
    y = jnp.dot(A, g, preferred_element_type=f32)
    out = dinv[:, None] * y + 2.0 * (dinv * dinv)[:, None] * h + b_ref[0, :][None, :]
    if relu:
        out = jnp.maximum(out, 0.0)
    o_ref[...] = out


def up_gcn(A, xs_j, u, perm, W, b, relu=True):
    n = A.shape[0]
    k = u.shape[0]
    return pl.pallas_call(
        functools.partial(_upgcn_body, relu),
        out_shape=jax.ShapeDtypeStruct((n, D), f32),
    )(A, xs_j, u, perm.reshape(1, k), W, b.reshape(1, D))


NPAD = 10016


def edge_stats(srcP, dstP):
    half = srcP.shape[0] // 2
    out = jnp.zeros((2, 2, NPAD), f32)
    for c in range(2):
        s = srcP[c * half:(c + 1) * half]
        d = dstP[c * half:(c + 1) * half]
        out = out.at[c, 0, :].add(jnp.zeros(NPAD, f32).at[d].add(1.0))
        out = out.at[c, 1, :].add(
            jnp.zeros(NPAD, f32).at[d].add(jnp.where(s == d, 1.0, 0.0)))
    return out


def spmm(g, srcP, dstP):
    half = srcP.shape[0] // 2
    out = jnp.zeros((2, NPAD, D), f32)
    for c in range(2):
        s = srcP[c * half:(c + 1) * half]
        d = dstP[c * half:(c + 1) * half]
        out = out.at[c].add(jnp.zeros((NPAD, D), f32).at[d].add(g[s]))
    return out


def build_restricted(srcB, dstB, isid, perm, npad, kpad):
    k = perm.shape[0]
    rank = jnp.full(npad, -1, i32).at[perm].set(jnp.arange(k, dtype=i32))
    rs = rank[srcB]
    rd = rank[dstB]
    keep = isid | (srcB != dstB)
    mB = (rs >= 0) & keep
    mC = (rd >= 0) & keep
    B = jnp.zeros((npad, kpad), f32).at[dstB, jnp.clip(rs, 0)].add(
        jnp.where(mB, 1.0, 0.0))
    CT = jnp.zeros((npad, kpad), f32).at[srcB, jnp.clip(rd, 0)].add(
        jnp.where(mC, 1.0, 0.0))
    return B, CT


def gather_rows(x, permP):
    return x[permP]


def scatter_add_rows(base, perm, u):
    return base.at[perm].add(u)


def kernel(x, edge_index, W0, b0, W1, b1, W2, b2, W3, b3,
           U0, c0, U1, c1, U2, c2, p1, p2, p3):
    src = edge_index[0]
    dst = edge_index[1]

    EPT = 10112
    E_pad = 32 * EPT
    padn = E_pad - E
    srcP = jnp.concatenate([src, jnp.zeros(padn, i32)])
    dstP = jnp.concatenate([dst, jnp.full(padn, N, i32)])


    stats = edge_stats(srcP, dstP)
    g0, dinv0, selfw0 = stats_combine_premul(stats, x, W0)
    y0 = spmm(g0, srcP, dstP)
    pn1 = p1 / jnp.sqrt(jnp.sum(p1 * p1))
    x1, s1 = gcn0_post(y0, g0, dinv0, selfw0, b0, pn1)

    vals1, perm1 = lax.top_k(s1[:, 0], KS[0])

    NPAD2 = 10240
    KPAD = 2048
    padb = E_pad - E - KS[0]
    srcB = jnp.concatenate([src, perm1, jnp.zeros(padb, i32)])
    dstB = jnp.concatenate([dst, perm1, jnp.full(padb, 1, i32)])
    isid = jnp.arange(E_pad, dtype=i32)
    isid = (isid >= E) & (isid < E + KS[0])
    B, CT = build_restricted(srcB, dstB, isid, perm1, NPAD2, KPAD)
    Ap1 = bigmm_diag0(CT, B, NPAD2, KPAD)[:KS[0], :KS[0]]

    permP1 = jnp.concatenate([perm1, jnp.zeros(48, i32)])
    xp1 = gather_rows(x1, permP1)[:KS[0]]

    pn2 = p2 / jnp.sqrt(jnp.sum(p2 * p2))
    x2, s2 = gcn_dense(Ap1, xp1, W1, b1, vals=vals1.reshape(-1, 1), pn=pn2)
    vals2, perm2 = lax.top_k(s2[:, 0], KS[1])

    Ap2, xp2 = aug_pool(Ap1, perm2, x2, vals2)
    pn3 = p3 / jnp.sqrt(jnp.sum(p3 * p3))
    x3, s3 = gcn_dense(Ap2, xp2, W2, b2, pn=pn3)
    vals3, perm3 = lax.top_k(s3[:, 0], KS[2])

    Ap3, xp3 = aug_pool(Ap2, perm3, x3, vals3)
    x4, _ = gcn_dense(Ap3, xp3, W3, b3)

    u = up_gcn(Ap2, x3, x4, perm3, U0, c0, relu=True)
    u = up_gcn(Ap1, x2, u, perm2, U1, c1, relu=True)

    xU = scatter_add_rows(x1, perm1, u)
    gU = premul(xU, U2, dinv0)
    yU = spmm(gU, srcP, dstP)
    out = final_post(yU, gU, dinv0, selfw0, c2)
    return out
